# baseline (speedup 1.0000x reference)
import functools

import jax
import jax.numpy as jnp
from jax import lax
from jax.experimental import pallas as pl
from jax.experimental.pallas import tpu as pltpu
from jax.experimental.pallas import tpu_sc as plsc

N = 10000
E = 320000
N_PAD = 10240
NTILES = 32
ZR = N_PAD // 16
CHUNK = 128
NBUF = 8
CPW = NBUF * (-(-E // (NTILES * CHUNK * NBUF)))
E_PAD = NTILES * CHUNK * CPW
NCH = E_PAD // CHUNK
CPW0 = NCH // 16
_SLABC = 64
_PARTS = (64, 64, 32)

_f32 = jnp.float32
_i32 = jnp.int32


def _sc_mesh():
    return plsc.VectorSubcoreMesh(core_axis_name="c", subcore_axis_name="s")


@functools.partial(jax.jit, static_argnames=("d",))
def _sc_propagate(g, zeros, src, dst, *, d):
    @functools.partial(
        pl.kernel,
        out_type=jax.ShapeDtypeStruct((N_PAD, d), _f32),
        mesh=_sc_mesh(),
        scratch_types=(
            [pltpu.VMEM((_SLABC, CHUNK), _i32),
             pltpu.VMEM((_SLABC, CHUNK), _i32),
             pltpu.VMEM((NBUF, CHUNK, d), _f32),
             pltpu.VMEM_SHARED((N_PAD, d), _f32)]
            + [pltpu.SemaphoreType.DMA] * NBUF
        ),
        compiler_params=pltpu.CompilerParams(use_tc_tiling_on_sc=False),
    )
    def prop(g_hbm, z_hbm, src_hbm, dst_hbm, out_hbm,
             src_v, dst_v, rows, acc, *sems):
        cid = lax.axis_index("c")
        sid = lax.axis_index("s")

        @pl.when(cid == 0)
        def _zero():
            pltpu.sync_copy(z_hbm, acc.at[pl.ds(sid * ZR, ZR)])
        plsc.subcore_barrier()

        def part(base, cpw):
            pltpu.sync_copy(src_hbm.at[pl.ds(base, cpw)],
                            src_v.at[pl.ds(0, cpw)])
            pltpu.sync_copy(dst_hbm.at[pl.ds(base, cpw)],
                            dst_v.at[pl.ds(0, cpw)])
            for b in range(NBUF):
                pltpu.async_copy(g_hbm.at[src_v.at[b]], rows.at[b], sems[b])

            @pl.loop(0, cpw // NBUF)
            def _group(k):
                c0 = k * NBUF
                for b in range(NBUF):
                    c = c0 + b
                    pltpu.make_async_copy(
                        g_hbm.at[src_v.at[c]], rows.at[b], sems[b]).wait()
                    pltpu.sync_copy(rows.at[b], acc.at[dst_v.at[c]], add=True)

                    @pl.when(c + NBUF < cpw)
                    def _refill():
                        pltpu.async_copy(
                            g_hbm.at[src_v.at[c + NBUF]], rows.at[b], sems[b])

        @pl.when(cid == 0)
        def _core0():
            off = 0
            for sz in _PARTS:
                part(sid * CPW0 + off, sz)
                off += sz

        plsc.subcore_barrier()

        @pl.when(cid == 0)
        def _writeback():
            pltpu.sync_copy(acc.at[pl.ds(sid * ZR, ZR)],
                            out_hbm.at[pl.ds(sid * ZR, ZR)])

    return prop(g, zeros, src, dst)


_SLAB = NBUF * CHUNK


@jax.jit
def _sc_edge_gather(h, src, dst):
    @functools.partial(
        pl.kernel,
        out_type=jax.ShapeDtypeStruct((2, E_PAD, 16), _f32),
        mesh=_sc_mesh(),
        scratch_types=(
            [pltpu.VMEM((CPW0, CHUNK), _i32),
             pltpu.VMEM((CPW0, CHUNK), _i32),
             pltpu.VMEM((2, _SLAB, 16), _f32),
             pltpu.VMEM((2, _SLAB, 16), _f32)]
            + [pltpu.SemaphoreType.DMA] * 8
        ),
        name="egather",
        compiler_params=pltpu.CompilerParams(use_tc_tiling_on_sc=False),
    )
    def egather(h_hbm, src_hbm, dst_hbm, out_hbm,
                src_v, dst_v, slab_s, slab_d,
                sgs0, sgs1, sgd0, sgd1, sws0, sws1, swd0, swd1):
        sgs, sgd = (sgs0, sgs1), (sgd0, sgd1)
        sws, swd = (sws0, sws1), (swd0, swd1)
        cid = lax.axis_index("c")
        sid = lax.axis_index("s")

        def run(base_chunk, cnt):
            pltpu.sync_copy(src_hbm.at[pl.ds(base_chunk, cnt)],
                            src_v.at[pl.ds(0, cnt)])
            pltpu.sync_copy(dst_hbm.at[pl.ds(base_chunk, cnt)],
                            dst_v.at[pl.ds(0, cnt)])
            ng = cnt // NBUF
            base = base_chunk * CHUNK

            def fire_gathers(g, p):
                for b in range(NBUF):
                    c = g * NBUF + b
                    pltpu.async_copy(h_hbm.at[src_v.at[c]],
                                     slab_s.at[p, pl.ds(b * CHUNK, CHUNK)],
                                     sgs[p])
                    pltpu.async_copy(h_hbm.at[dst_v.at[c]],
                                     slab_d.at[p, pl.ds(b * CHUNK, CHUNK)],
                                     sgd[p])

            def drain_gathers(g, p):
                for b in range(NBUF):
                    c = g * NBUF + b
                    pltpu.make_async_copy(
                        h_hbm.at[src_v.at[c]],
                        slab_s.at[p, pl.ds(b * CHUNK, CHUNK)], sgs[p]).wait()
                    pltpu.make_async_copy(
                        h_hbm.at[dst_v.at[c]],
                        slab_d.at[p, pl.ds(b * CHUNK, CHUNK)], sgd[p]).wait()

            def fire_writes(g, p):
                eb = base + g * _SLAB
                pltpu.async_copy(slab_s.at[p],
                                 out_hbm.at[0, pl.ds(eb, _SLAB)], sws[p])
                pltpu.async_copy(slab_d.at[p],
                                 out_hbm.at[1, pl.ds(eb, _SLAB)], swd[p])

            def drain_writes(g, p):
                eb = base + g * _SLAB
                pltpu.make_async_copy(
                    slab_s.at[p], out_hbm.at[0, pl.ds(eb, _SLAB)], sws[p]).wait()
                pltpu.make_async_copy(
                    slab_d.at[p], out_hbm.at[1, pl.ds(eb, _SLAB)], swd[p]).wait()

            fire_gathers(0, 0)

            @pl.loop(0, ng, step=2)
            def _pair(k):
                @pl.when(k >= 2)
                def _():
                    drain_writes(k - 1, 1)
                fire_gathers(k + 1, 1)
                drain_gathers(k, 0)
                fire_writes(k, 0)

                @pl.when(k + 2 < ng)
                def _():
                    drain_writes(k, 0)
                    fire_gathers(k + 2, 0)
                drain_gathers(k + 1, 1)
                fire_writes(k + 1, 1)

            drain_writes(ng - 2, 0)
            drain_writes(ng - 1, 1)

        @pl.when(cid == 0)
        def _core0():
            run(sid * CPW0, CPW0)

    return egather(h, src, dst)


_BN = 1024
_GN = N_PAD // _BN


def _tc_head(degp, x, W1):
    def body(p_ref, x_ref, w_ref, dinv_ref, g_ref):
        deg = p_ref[:, 0:1] + 1.0
        dv = lax.rsqrt(deg)
        dinv_ref[...] = dv
        g_ref[...] = (x_ref[...] @ w_ref[...]) * dv

    return pl.pallas_call(
        body,
        grid=(_GN,),
        in_specs=[
            pl.BlockSpec((_BN, 16), lambda i: (i, 0)),
            pl.BlockSpec((_BN, 128), lambda i: (i, 0)),
            pl.BlockSpec((128, 64), lambda i: (0, 0)),
        ],
        out_specs=[
            pl.BlockSpec((_BN, 1), lambda i: (i, 0)),
            pl.BlockSpec((_BN, 64), lambda i: (i, 0)),
        ],
        out_shape=[
            jax.ShapeDtypeStruct((N_PAD, 1), _f32),
            jax.ShapeDtypeStruct((N_PAD, 64), _f32),
        ],
    )(degp, x, W1)


def _tc_combine_transform(p, g, dinv, b, W):
    d_in = g.shape[1]
    d_out = W.shape[1]

    def body(p_ref, g_ref, dv_ref, b_ref, w_ref, o_ref):
        dv = dv_ref[...]
        z = (p_ref[...] + g_ref[...]) * dv + b_ref[...]
        o_ref[...] = (jnp.tanh(z) @ w_ref[...]) * dv

    return pl.pallas_call(
        body,
        grid=(_GN,),
        in_specs=[
            pl.BlockSpec((_BN, d_in), lambda i: (i, 0)),
            pl.BlockSpec((_BN, d_in), lambda i: (i, 0)),
            pl.BlockSpec((_BN, 1), lambda i: (i, 0)),
            pl.BlockSpec((1, d_in), lambda i: (0, 0)),
            pl.BlockSpec((d_in, d_out), lambda i: (0, 0)),
        ],
        out_specs=pl.BlockSpec((_BN, d_out), lambda i: (i, 0)),
        out_shape=jax.ShapeDtypeStruct((N_PAD, d_out), _f32),
    )(p, g, dinv, b, W)


def _tc_combine_only(p, g, dinv, b):
    d_in = g.shape[1]

    def body(p_ref, g_ref, dv_ref, b_ref, o_ref):
        dv = dv_ref[...]
        z = (p_ref[...] + g_ref[...]) * dv + b_ref[...]
        o_ref[...] = jnp.tanh(z) * dv

    return pl.pallas_call(
        body,
        grid=(_GN,),
        in_specs=[
            pl.BlockSpec((_BN, d_in), lambda i: (i, 0)),
            pl.BlockSpec((_BN, d_in), lambda i: (i, 0)),
            pl.BlockSpec((_BN, 1), lambda i: (i, 0)),
            pl.BlockSpec((1, d_in), lambda i: (0, 0)),
        ],
        out_specs=pl.BlockSpec((_BN, d_in), lambda i: (i, 0)),
        out_shape=jax.ShapeDtypeStruct((N_PAD, d_in), _f32),
    )(p, g, dinv, b)


def _tc_tail(p, g5, dinv, W5, b5, lW1p, lb1p, lW2p, lb2p, lW3p, lb3p):
    def body(p_ref, g_ref, dv_ref, w5_ref, b5_ref,
             w1_ref, c1_ref, w2_ref, c2_ref, w3_ref, c3_ref, o_ref):
        q = (p_ref[...] + g_ref[...]) * dv_ref[...]
        h5 = jnp.tanh(q @ w5_ref[...] + b5_ref[...])
        t1 = jnp.tanh(h5 @ w1_ref[...] + c1_ref[...])
        t2 = jnp.tanh(t1 @ w2_ref[...] + c2_ref[...])
        o_ref[...] = jnp.tanh(t2 @ w3_ref[...] + c3_ref[...])

    return pl.pallas_call(
        body,
        grid=(_GN,),
        in_specs=[
            pl.BlockSpec((_BN, 32), lambda i: (i, 0)),
            pl.BlockSpec((_BN, 32), lambda i: (i, 0)),
            pl.BlockSpec((_BN, 1), lambda i: (i, 0)),
            pl.BlockSpec((32, 256), lambda i: (0, 0)),
            pl.BlockSpec((1, 256), lambda i: (0, 0)),
            pl.BlockSpec((256, 32), lambda i: (0, 0)),
            pl.BlockSpec((1, 32), lambda i: (0, 0)),
            pl.BlockSpec((32, 32), lambda i: (0, 0)),
            pl.BlockSpec((1, 32), lambda i: (0, 0)),
            pl.BlockSpec((32, 16), lambda i: (0, 0)),
            pl.BlockSpec((1, 16), lambda i: (0, 0)),
        ],
        out_specs=pl.BlockSpec((_BN, 16), lambda i: (i, 0)),
        out_shape=jax.ShapeDtypeStruct((N_PAD, 16), _f32),
    )(p, g5, dinv, W5, b5, lW1p, lb1p, lW2p, lb2p, lW3p, lb3p)


_BE = 3200
_GE = E // _BE


def _tc_edge(he, cWa, cWb, cb):
    def body(hs_ref, hd_ref, wa_ref, wb_ref, cb_ref, out_ref, e_ref):
        hs = hs_ref[0]
        hd = hd_ref[0]
        e_ref[...] = jnp.concatenate([hs[:, :12], hd[:, :12]], axis=1)
        out_ref[...] = hs @ wa_ref[...] + hd @ wb_ref[...] + cb_ref[...]

    return pl.pallas_call(
        body,
        grid=(_GE,),
        in_specs=[
            pl.BlockSpec((1, _BE, 16), lambda i: (0, i, 0)),
            pl.BlockSpec((1, _BE, 16), lambda i: (1, i, 0)),
            pl.BlockSpec((16, 40), lambda i: (0, 0)),
            pl.BlockSpec((16, 40), lambda i: (0, 0)),
            pl.BlockSpec((1, 40), lambda i: (0, 0)),
        ],
        out_specs=[
            pl.BlockSpec((_BE, 40), lambda i: (i, 0)),
            pl.BlockSpec((_BE, 24), lambda i: (i, 0)),
        ],
        out_shape=[
            jax.ShapeDtypeStruct((E, 40), _f32),
            jax.ShapeDtypeStruct((E, 24), _f32),
        ],
    )(he, he, cWa, cWb, cb)


def kernel(x, edge_index, batch, W1, b1, W2, b2, W3, b3, W4, b4, W5, b5,
           lW1, lb1, lW2, lb2, lW3, lb3, cW, cb):
    pad_e = E_PAD - E
    src = jnp.concatenate(
        [edge_index[0], jnp.full((pad_e,), N_PAD - 1, _i32)]).reshape(NCH, CHUNK)
    dst = jnp.concatenate(
        [edge_index[1], jnp.full((pad_e,), N_PAD - 1, _i32)]).reshape(NCH, CHUNK)
    x_p = jnp.pad(x, ((0, N_PAD - N), (0, 0)))

    z16 = jnp.zeros((ZR, 16), _f32)
    z32 = jnp.zeros((ZR, 32), _f32)
    z64 = jnp.zeros((ZR, 64), _f32)
    e1 = jnp.zeros((N_PAD, 16), _f32).at[:, 0].set(1.0)

    lW1p = jnp.pad(lW1, ((0, 0), (0, 8)))
    lb1p = jnp.pad(lb1, (0, 8)).reshape(1, 32)
    lW2p = jnp.pad(lW2, ((0, 8), (0, 14)))
    lb2p = jnp.pad(lb2, (0, 14)).reshape(1, 32)
    lW3p = jnp.pad(lW3, ((0, 14), (0, 4)))
    lb3p = jnp.pad(lb3, (0, 4)).reshape(1, 16)
    cWa = jnp.pad(cW[:12], ((0, 4), (0, 0)))
    cWb = jnp.pad(cW[12:], ((0, 4), (0, 0)))
    cbr = cb.reshape(1, 40)

    degp = _sc_propagate(e1, z16, src, dst, d=16)

    dinv, g1 = _tc_head(degp, x_p, W1)
    p = _sc_propagate(g1, z64, src, dst, d=64)

    g2 = _tc_combine_transform(p, g1, dinv, b1.reshape(1, 64), W2)
    p = _sc_propagate(g2, z64, src, dst, d=64)
    g3 = _tc_combine_transform(p, g2, dinv, b2.reshape(1, 64), W3)
    p = _sc_propagate(g3, z32, src, dst, d=32)
    g4 = _tc_combine_transform(p, g3, dinv, b3.reshape(1, 32), W4)
    p = _sc_propagate(g4, z32, src, dst, d=32)

    g5 = _tc_combine_only(p, g4, dinv, b4.reshape(1, 32))
    p = _sc_propagate(g5, z32, src, dst, d=32)
    hf = _tc_tail(p, g5, dinv, W5, b5.reshape(1, 256),
                  lW1p, lb1p, lW2p, lb2p, lW3p, lb3p)

    he = _sc_edge_gather(hf, src, dst)
    out, e = _tc_edge(he, cWa, cWb, cbr)
    return (out, e)


# speedup vs baseline: 1.2083x; 1.2083x over previous
import functools

import jax
import jax.numpy as jnp
from jax import lax
from jax.experimental import pallas as pl
from jax.experimental.pallas import tpu as pltpu
from jax.experimental.pallas import tpu_sc as plsc

N = 10000
E = 320000
N_PAD = 10240
NTILES = 32
ZR = N_PAD // 16
CHUNK = 128
NBUF = 8
CPW = NBUF * (-(-E // (NTILES * CHUNK * NBUF)))
E_PAD = NTILES * CHUNK * CPW
NCH = E_PAD // CHUNK
CPW0 = 128
CPW1 = 32
_SLABC = 64

_f32 = jnp.float32
_i32 = jnp.int32


def _sc_mesh():
    return plsc.VectorSubcoreMesh(core_axis_name="c", subcore_axis_name="s")


@functools.partial(jax.jit, static_argnames=("d",))
def _sc_propagate(g, zeros, src, dst, *, d):
    @functools.partial(
        pl.kernel,
        out_type=jax.ShapeDtypeStruct((2, N_PAD, d), _f32),
        mesh=_sc_mesh(),
        scratch_types=(
            [pltpu.VMEM((_SLABC, CHUNK), _i32),
             pltpu.VMEM((_SLABC, CHUNK), _i32),
             pltpu.VMEM((NBUF, CHUNK, d), _f32),
             pltpu.VMEM_SHARED((N_PAD, d), _f32)]
            + [pltpu.SemaphoreType.DMA] * NBUF
        ),
        compiler_params=pltpu.CompilerParams(use_tc_tiling_on_sc=False),
    )
    def prop(g_hbm, z_hbm, src_hbm, dst_hbm, out_hbm,
             src_v, dst_v, rows, acc, *sems):
        cid = lax.axis_index("c")
        sid = lax.axis_index("s")
        pltpu.sync_copy(z_hbm, acc.at[pl.ds(sid * ZR, ZR)])
        plsc.subcore_barrier()

        def part(base, cpw):
            pltpu.sync_copy(src_hbm.at[pl.ds(base, cpw)],
                            src_v.at[pl.ds(0, cpw)])
            pltpu.sync_copy(dst_hbm.at[pl.ds(base, cpw)],
                            dst_v.at[pl.ds(0, cpw)])
            for b in range(NBUF):
                pltpu.async_copy(g_hbm.at[src_v.at[b]], rows.at[b], sems[b])

            @pl.loop(0, cpw // NBUF)
            def _group(k):
                c0 = k * NBUF
                for b in range(NBUF):
                    c = c0 + b
                    pltpu.make_async_copy(
                        g_hbm.at[src_v.at[c]], rows.at[b], sems[b]).wait()
                    pltpu.sync_copy(rows.at[b], acc.at[dst_v.at[c]], add=True)

                    @pl.when(c + NBUF < cpw)
                    def _refill():
                        pltpu.async_copy(
                            g_hbm.at[src_v.at[c + NBUF]], rows.at[b], sems[b])

        @pl.when(cid == 0)
        def _core0():
            for h in range(CPW0 // _SLABC):
                part(sid * CPW0 + h * _SLABC, _SLABC)

        @pl.when(cid == 1)
        def _core1():
            part(16 * CPW0 + sid * CPW1, CPW1)

        plsc.subcore_barrier()
        pltpu.sync_copy(acc.at[pl.ds(sid * ZR, ZR)],
                        out_hbm.at[cid, pl.ds(sid * ZR, ZR)])

    return prop(g, zeros, src, dst)


_SLAB = NBUF * CHUNK


@jax.jit
def _sc_edge_gather(h, src, dst):
    @functools.partial(
        pl.kernel,
        out_type=jax.ShapeDtypeStruct((2, E_PAD, 16), _f32),
        mesh=_sc_mesh(),
        scratch_types=(
            [pltpu.VMEM((CPW0, CHUNK), _i32),
             pltpu.VMEM((CPW0, CHUNK), _i32),
             pltpu.VMEM((2, _SLAB, 16), _f32),
             pltpu.VMEM((2, _SLAB, 16), _f32)]
            + [pltpu.SemaphoreType.DMA] * 8
        ),
        compiler_params=pltpu.CompilerParams(use_tc_tiling_on_sc=False),
    )
    def egather(h_hbm, src_hbm, dst_hbm, out_hbm,
                src_v, dst_v, slab_s, slab_d,
                sgs0, sgs1, sgd0, sgd1, sws0, sws1, swd0, swd1):
        sgs, sgd = (sgs0, sgs1), (sgd0, sgd1)
        sws, swd = (sws0, sws1), (swd0, swd1)
        cid = lax.axis_index("c")
        sid = lax.axis_index("s")

        def run(base_chunk, cnt):
            pltpu.sync_copy(src_hbm.at[pl.ds(base_chunk, cnt)],
                            src_v.at[pl.ds(0, cnt)])
            pltpu.sync_copy(dst_hbm.at[pl.ds(base_chunk, cnt)],
                            dst_v.at[pl.ds(0, cnt)])
            ng = cnt // NBUF
            base = base_chunk * CHUNK

            def fire_gathers(g, p):
                for b in range(NBUF):
                    c = g * NBUF + b
                    pltpu.async_copy(h_hbm.at[src_v.at[c]],
                                     slab_s.at[p, pl.ds(b * CHUNK, CHUNK)],
                                     sgs[p])
                    pltpu.async_copy(h_hbm.at[dst_v.at[c]],
                                     slab_d.at[p, pl.ds(b * CHUNK, CHUNK)],
                                     sgd[p])

            def drain_gathers(g, p):
                for b in range(NBUF):
                    c = g * NBUF + b
                    pltpu.make_async_copy(
                        h_hbm.at[src_v.at[c]],
                        slab_s.at[p, pl.ds(b * CHUNK, CHUNK)], sgs[p]).wait()
                    pltpu.make_async_copy(
                        h_hbm.at[dst_v.at[c]],
                        slab_d.at[p, pl.ds(b * CHUNK, CHUNK)], sgd[p]).wait()

            def fire_writes(g, p):
                eb = base + g * _SLAB
                pltpu.async_copy(slab_s.at[p],
                                 out_hbm.at[0, pl.ds(eb, _SLAB)], sws[p])
                pltpu.async_copy(slab_d.at[p],
                                 out_hbm.at[1, pl.ds(eb, _SLAB)], swd[p])

            def drain_writes(g, p):
                eb = base + g * _SLAB
                pltpu.make_async_copy(
                    slab_s.at[p], out_hbm.at[0, pl.ds(eb, _SLAB)], sws[p]).wait()
                pltpu.make_async_copy(
                    slab_d.at[p], out_hbm.at[1, pl.ds(eb, _SLAB)], swd[p]).wait()

            fire_gathers(0, 0)

            @pl.loop(0, ng, step=2)
            def _pair(k):
                @pl.when(k >= 2)
                def _():
                    drain_writes(k - 1, 1)
                fire_gathers(k + 1, 1)
                drain_gathers(k, 0)
                fire_writes(k, 0)

                @pl.when(k + 2 < ng)
                def _():
                    drain_writes(k, 0)
                    fire_gathers(k + 2, 0)
                drain_gathers(k + 1, 1)
                fire_writes(k + 1, 1)

            drain_writes(ng - 2, 0)
            drain_writes(ng - 1, 1)

        @pl.when(cid == 0)
        def _core0():
            run(sid * CPW0, CPW0)

        @pl.when(cid == 1)
        def _core1():
            run(16 * CPW0 + sid * CPW1, CPW1)

    return egather(h, src, dst)


_BN = 1024
_GN = N_PAD // _BN


def _tc_head(degp, x, W1):
    def body(p_ref, x_ref, w_ref, dinv_ref, g_ref):
        deg = p_ref[0][:, 0:1] + p_ref[1][:, 0:1] + 1.0
        dv = lax.rsqrt(deg)
        dinv_ref[...] = dv
        g_ref[...] = (x_ref[...] @ w_ref[...]) * dv

    return pl.pallas_call(
        body,
        grid=(_GN,),
        in_specs=[
            pl.BlockSpec((2, _BN, 16), lambda i: (0, i, 0)),
            pl.BlockSpec((_BN, 128), lambda i: (i, 0)),
            pl.BlockSpec((128, 64), lambda i: (0, 0)),
        ],
        out_specs=[
            pl.BlockSpec((_BN, 1), lambda i: (i, 0)),
            pl.BlockSpec((_BN, 64), lambda i: (i, 0)),
        ],
        out_shape=[
            jax.ShapeDtypeStruct((N_PAD, 1), _f32),
            jax.ShapeDtypeStruct((N_PAD, 64), _f32),
        ],
    )(degp, x, W1)


def _tc_combine_transform(p, g, dinv, b, W):
    d_in = g.shape[1]
    d_out = W.shape[1]

    def body(p_ref, g_ref, dv_ref, b_ref, w_ref, o_ref):
        dv = dv_ref[...]
        z = (p_ref[0] + p_ref[1] + g_ref[...]) * dv + b_ref[...]
        o_ref[...] = (jnp.tanh(z) @ w_ref[...]) * dv

    return pl.pallas_call(
        body,
        grid=(_GN,),
        in_specs=[
            pl.BlockSpec((2, _BN, d_in), lambda i: (0, i, 0)),
            pl.BlockSpec((_BN, d_in), lambda i: (i, 0)),
            pl.BlockSpec((_BN, 1), lambda i: (i, 0)),
            pl.BlockSpec((1, d_in), lambda i: (0, 0)),
            pl.BlockSpec((d_in, d_out), lambda i: (0, 0)),
        ],
        out_specs=pl.BlockSpec((_BN, d_out), lambda i: (i, 0)),
        out_shape=jax.ShapeDtypeStruct((N_PAD, d_out), _f32),
    )(p, g, dinv, b, W)


def _tc_combine_only(p, g, dinv, b):
    d_in = g.shape[1]

    def body(p_ref, g_ref, dv_ref, b_ref, o_ref):
        dv = dv_ref[...]
        z = (p_ref[0] + p_ref[1] + g_ref[...]) * dv + b_ref[...]
        o_ref[...] = jnp.tanh(z) * dv

    return pl.pallas_call(
        body,
        grid=(_GN,),
        in_specs=[
            pl.BlockSpec((2, _BN, d_in), lambda i: (0, i, 0)),
            pl.BlockSpec((_BN, d_in), lambda i: (i, 0)),
            pl.BlockSpec((_BN, 1), lambda i: (i, 0)),
            pl.BlockSpec((1, d_in), lambda i: (0, 0)),
        ],
        out_specs=pl.BlockSpec((_BN, d_in), lambda i: (i, 0)),
        out_shape=jax.ShapeDtypeStruct((N_PAD, d_in), _f32),
    )(p, g, dinv, b)


def _tc_tail(p, g5, dinv, W5, b5, lW1p, lb1p, lW2p, lb2p, lW3p, lb3p):
    def body(p_ref, g_ref, dv_ref, w5_ref, b5_ref,
             w1_ref, c1_ref, w2_ref, c2_ref, w3_ref, c3_ref, o_ref):
        q = (p_ref[0] + p_ref[1] + g_ref[...]) * dv_ref[...]
        h5 = jnp.tanh(q @ w5_ref[...] + b5_ref[...])
        t1 = jnp.tanh(h5 @ w1_ref[...] + c1_ref[...])
        t2 = jnp.tanh(t1 @ w2_ref[...] + c2_ref[...])
        o_ref[...] = jnp.tanh(t2 @ w3_ref[...] + c3_ref[...])

    return pl.pallas_call(
        body,
        grid=(_GN,),
        in_specs=[
            pl.BlockSpec((2, _BN, 32), lambda i: (0, i, 0)),
            pl.BlockSpec((_BN, 32), lambda i: (i, 0)),
            pl.BlockSpec((_BN, 1), lambda i: (i, 0)),
            pl.BlockSpec((32, 256), lambda i: (0, 0)),
            pl.BlockSpec((1, 256), lambda i: (0, 0)),
            pl.BlockSpec((256, 32), lambda i: (0, 0)),
            pl.BlockSpec((1, 32), lambda i: (0, 0)),
            pl.BlockSpec((32, 32), lambda i: (0, 0)),
            pl.BlockSpec((1, 32), lambda i: (0, 0)),
            pl.BlockSpec((32, 16), lambda i: (0, 0)),
            pl.BlockSpec((1, 16), lambda i: (0, 0)),
        ],
        out_specs=pl.BlockSpec((_BN, 16), lambda i: (i, 0)),
        out_shape=jax.ShapeDtypeStruct((N_PAD, 16), _f32),
    )(p, g5, dinv, W5, b5, lW1p, lb1p, lW2p, lb2p, lW3p, lb3p)


_BE = 3200
_GE = E // _BE


def _tc_edge(he, Wout, Se, cbt):
    def body(hs_ref, hd_ref, wo_ref, se_ref, cb_ref, out_ref, e_ref):
        hsd = jnp.concatenate([hs_ref[0], hd_ref[0]], axis=1)
        dn = (((1,), (1,)), ((), ()))
        out_ref[...] = lax.dot_general(wo_ref[...], hsd, dn) + cb_ref[...]
        e_ref[...] = lax.dot_general(se_ref[...], hsd, dn)

    return pl.pallas_call(
        body,
        grid=(_GE,),
        in_specs=[
            pl.BlockSpec((1, _BE, 16), lambda i: (0, i, 0)),
            pl.BlockSpec((1, _BE, 16), lambda i: (1, i, 0)),
            pl.BlockSpec((40, 32), lambda i: (0, 0)),
            pl.BlockSpec((24, 32), lambda i: (0, 0)),
            pl.BlockSpec((40, 1), lambda i: (0, 0)),
        ],
        out_specs=[
            pl.BlockSpec((40, _BE), lambda i: (0, i)),
            pl.BlockSpec((24, _BE), lambda i: (0, i)),
        ],
        out_shape=[
            jax.ShapeDtypeStruct((40, E), _f32),
            jax.ShapeDtypeStruct((24, E), _f32),
        ],
    )(he, he, Wout, Se, cbt)


def kernel(x, edge_index, batch, W1, b1, W2, b2, W3, b3, W4, b4, W5, b5,
           lW1, lb1, lW2, lb2, lW3, lb3, cW, cb):
    pad_e = E_PAD - E
    src = jnp.concatenate(
        [edge_index[0], jnp.full((pad_e,), N_PAD - 1, _i32)]).reshape(NCH, CHUNK)
    dst = jnp.concatenate(
        [edge_index[1], jnp.full((pad_e,), N_PAD - 1, _i32)]).reshape(NCH, CHUNK)
    x_p = jnp.pad(x, ((0, N_PAD - N), (0, 0)))

    z16 = jnp.zeros((ZR, 16), _f32)
    z32 = jnp.zeros((ZR, 32), _f32)
    z64 = jnp.zeros((ZR, 64), _f32)
    e1 = jnp.zeros((N_PAD, 16), _f32).at[:, 0].set(1.0)

    lW1p = jnp.pad(lW1, ((0, 0), (0, 8)))
    lb1p = jnp.pad(lb1, (0, 8)).reshape(1, 32)
    lW2p = jnp.pad(lW2, ((0, 8), (0, 14)))
    lb2p = jnp.pad(lb2, (0, 14)).reshape(1, 32)
    lW3p = jnp.pad(lW3, ((0, 14), (0, 4)))
    lb3p = jnp.pad(lb3, (0, 4)).reshape(1, 16)

    cWa = jnp.pad(cW[:12], ((0, 4), (0, 0)))
    cWb = jnp.pad(cW[12:], ((0, 4), (0, 0)))
    Wout = jnp.concatenate([cWa.T, cWb.T], axis=1)
    ii = jnp.arange(12)
    Se = (jnp.zeros((24, 32), _f32)
          .at[ii, ii].set(1.0)
          .at[12 + ii, 16 + ii].set(1.0))
    cbt = cb.reshape(40, 1)

    degp = _sc_propagate(e1, z16, src, dst, d=16)

    dinv, g1 = _tc_head(degp, x_p, W1)
    p = _sc_propagate(g1, z64, src, dst, d=64)

    g2 = _tc_combine_transform(p, g1, dinv, b1.reshape(1, 64), W2)
    p = _sc_propagate(g2, z64, src, dst, d=64)
    g3 = _tc_combine_transform(p, g2, dinv, b2.reshape(1, 64), W3)
    p = _sc_propagate(g3, z32, src, dst, d=32)
    g4 = _tc_combine_transform(p, g3, dinv, b3.reshape(1, 32), W4)
    p = _sc_propagate(g4, z32, src, dst, d=32)

    g5 = _tc_combine_only(p, g4, dinv, b4.reshape(1, 32))
    p = _sc_propagate(g5, z32, src, dst, d=32)
    hf = _tc_tail(p, g5, dinv, W5, b5.reshape(1, 256),
                  lW1p, lb1p, lW2p, lb2p, lW3p, lb3p)

    he = _sc_edge_gather(hf, src, dst)
    out_t, e_t = _tc_edge(he, Wout, Se, cbt)
    return (out_t.T, e_t.T)


# speedup vs baseline: 1.3274x; 1.0986x over previous
import functools

import jax
import jax.numpy as jnp
from jax import lax
from jax.experimental import pallas as pl
from jax.experimental.pallas import tpu as pltpu
from jax.experimental.pallas import tpu_sc as plsc

N = 10000
E = 320000
N_PAD = 10240
NTILES = 32
ZR = N_PAD // 16
CHUNK = 128
NBUF = 8
CPW = NBUF * (-(-E // (NTILES * CHUNK * NBUF)))
E_PAD = NTILES * CHUNK * CPW
NCH = E_PAD // CHUNK
CPW0 = 144
CPW1 = 16
_SLABC = 72

_f32 = jnp.float32
_i32 = jnp.int32


def _sc_mesh():
    return plsc.VectorSubcoreMesh(core_axis_name="c", subcore_axis_name="s")


@functools.partial(jax.jit, static_argnames=("d",))
def _sc_propagate(g, zeros, src, dst, *, d):
    @functools.partial(
        pl.kernel,
        out_type=jax.ShapeDtypeStruct((2, N_PAD, d), _f32),
        mesh=_sc_mesh(),
        scratch_types=(
            [pltpu.VMEM((_SLABC, CHUNK), _i32),
             pltpu.VMEM((_SLABC, CHUNK), _i32),
             pltpu.VMEM((NBUF, CHUNK, d), _f32),
             pltpu.VMEM_SHARED((N_PAD, d), _f32)]
            + [pltpu.SemaphoreType.DMA] * NBUF
        ),
        compiler_params=pltpu.CompilerParams(use_tc_tiling_on_sc=False),
    )
    def prop(g_hbm, z_hbm, src_hbm, dst_hbm, out_hbm,
             src_v, dst_v, rows, acc, *sems):
        cid = lax.axis_index("c")
        sid = lax.axis_index("s")
        pltpu.sync_copy(z_hbm, acc.at[pl.ds(sid * ZR, ZR)])
        plsc.subcore_barrier()

        def part(base, cpw):
            pltpu.sync_copy(src_hbm.at[pl.ds(base, cpw)],
                            src_v.at[pl.ds(0, cpw)])
            pltpu.sync_copy(dst_hbm.at[pl.ds(base, cpw)],
                            dst_v.at[pl.ds(0, cpw)])
            for b in range(NBUF):
                pltpu.async_copy(g_hbm.at[src_v.at[b]], rows.at[b], sems[b])

            @pl.loop(0, cpw // NBUF)
            def _group(k):
                c0 = k * NBUF
                for b in range(NBUF):
                    c = c0 + b
                    pltpu.make_async_copy(
                        g_hbm.at[src_v.at[c]], rows.at[b], sems[b]).wait()
                    pltpu.sync_copy(rows.at[b], acc.at[dst_v.at[c]], add=True)

                    @pl.when(c + NBUF < cpw)
                    def _refill():
                        pltpu.async_copy(
                            g_hbm.at[src_v.at[c + NBUF]], rows.at[b], sems[b])

        @pl.when(cid == 0)
        def _core0():
            for h in range(CPW0 // _SLABC):
                part(sid * CPW0 + h * _SLABC, _SLABC)

        @pl.when(cid == 1)
        def _core1():
            part(16 * CPW0 + sid * CPW1, CPW1)

        plsc.subcore_barrier()
        pltpu.sync_copy(acc.at[pl.ds(sid * ZR, ZR)],
                        out_hbm.at[cid, pl.ds(sid * ZR, ZR)])

    return prop(g, zeros, src, dst)


_SLAB = NBUF * CHUNK


@jax.jit
def _sc_edge_gather(h, src, dst):
    @functools.partial(
        pl.kernel,
        out_type=jax.ShapeDtypeStruct((2, E_PAD, 16), _f32),
        mesh=_sc_mesh(),
        scratch_types=(
            [pltpu.VMEM((CPW0, CHUNK), _i32),
             pltpu.VMEM((CPW0, CHUNK), _i32),
             pltpu.VMEM((2, _SLAB, 16), _f32),
             pltpu.VMEM((2, _SLAB, 16), _f32)]
            + [pltpu.SemaphoreType.DMA] * 8
        ),
        compiler_params=pltpu.CompilerParams(use_tc_tiling_on_sc=False),
    )
    def egather(h_hbm, src_hbm, dst_hbm, out_hbm,
                src_v, dst_v, slab_s, slab_d,
                sgs0, sgs1, sgd0, sgd1, sws0, sws1, swd0, swd1):
        sgs, sgd = (sgs0, sgs1), (sgd0, sgd1)
        sws, swd = (sws0, sws1), (swd0, swd1)
        cid = lax.axis_index("c")
        sid = lax.axis_index("s")

        def run(base_chunk, cnt):
            pltpu.sync_copy(src_hbm.at[pl.ds(base_chunk, cnt)],
                            src_v.at[pl.ds(0, cnt)])
            pltpu.sync_copy(dst_hbm.at[pl.ds(base_chunk, cnt)],
                            dst_v.at[pl.ds(0, cnt)])
            ng = cnt // NBUF
            base = base_chunk * CHUNK

            def fire_gathers(g, p):
                for b in range(NBUF):
                    c = g * NBUF + b
                    pltpu.async_copy(h_hbm.at[src_v.at[c]],
                                     slab_s.at[p, pl.ds(b * CHUNK, CHUNK)],
                                     sgs[p])
                    pltpu.async_copy(h_hbm.at[dst_v.at[c]],
                                     slab_d.at[p, pl.ds(b * CHUNK, CHUNK)],
                                     sgd[p])

            def drain_gathers(g, p):
                for b in range(NBUF):
                    c = g * NBUF + b
                    pltpu.make_async_copy(
                        h_hbm.at[src_v.at[c]],
                        slab_s.at[p, pl.ds(b * CHUNK, CHUNK)], sgs[p]).wait()
                    pltpu.make_async_copy(
                        h_hbm.at[dst_v.at[c]],
                        slab_d.at[p, pl.ds(b * CHUNK, CHUNK)], sgd[p]).wait()

            def fire_writes(g, p):
                eb = base + g * _SLAB
                pltpu.async_copy(slab_s.at[p],
                                 out_hbm.at[0, pl.ds(eb, _SLAB)], sws[p])
                pltpu.async_copy(slab_d.at[p],
                                 out_hbm.at[1, pl.ds(eb, _SLAB)], swd[p])

            def drain_writes(g, p):
                eb = base + g * _SLAB
                pltpu.make_async_copy(
                    slab_s.at[p], out_hbm.at[0, pl.ds(eb, _SLAB)], sws[p]).wait()
                pltpu.make_async_copy(
                    slab_d.at[p], out_hbm.at[1, pl.ds(eb, _SLAB)], swd[p]).wait()

            fire_gathers(0, 0)

            @pl.loop(0, ng, step=2)
            def _pair(k):
                @pl.when(k >= 2)
                def _():
                    drain_writes(k - 1, 1)
                fire_gathers(k + 1, 1)
                drain_gathers(k, 0)
                fire_writes(k, 0)

                @pl.when(k + 2 < ng)
                def _():
                    drain_writes(k, 0)
                    fire_gathers(k + 2, 0)
                drain_gathers(k + 1, 1)
                fire_writes(k + 1, 1)

            drain_writes(ng - 2, 0)
            drain_writes(ng - 1, 1)

        @pl.when(cid == 0)
        def _core0():
            run(sid * CPW0, CPW0)

        @pl.when(cid == 1)
        def _core1():
            run(16 * CPW0 + sid * CPW1, CPW1)

    return egather(h, src, dst)


_BN = 1024
_GN = N_PAD // _BN


def _tc_head(degp, x, W1):
    def body(p_ref, x_ref, w_ref, dinv_ref, g_ref):
        deg = p_ref[0][:, 0:1] + p_ref[1][:, 0:1] + 1.0
        dv = lax.rsqrt(deg)
        dinv_ref[...] = dv
        g_ref[...] = (x_ref[...] @ w_ref[...]) * dv

    return pl.pallas_call(
        body,
        grid=(_GN,),
        in_specs=[
            pl.BlockSpec((2, _BN, 16), lambda i: (0, i, 0)),
            pl.BlockSpec((_BN, 128), lambda i: (i, 0)),
            pl.BlockSpec((128, 64), lambda i: (0, 0)),
        ],
        out_specs=[
            pl.BlockSpec((_BN, 1), lambda i: (i, 0)),
            pl.BlockSpec((_BN, 64), lambda i: (i, 0)),
        ],
        out_shape=[
            jax.ShapeDtypeStruct((N_PAD, 1), _f32),
            jax.ShapeDtypeStruct((N_PAD, 64), _f32),
        ],
    )(degp, x, W1)


def _tc_combine_transform(p, g, dinv, b, W):
    d_in = g.shape[1]
    d_out = W.shape[1]

    def body(p_ref, g_ref, dv_ref, b_ref, w_ref, o_ref):
        dv = dv_ref[...]
        z = (p_ref[0] + p_ref[1] + g_ref[...]) * dv + b_ref[...]
        o_ref[...] = (jnp.tanh(z) @ w_ref[...]) * dv

    return pl.pallas_call(
        body,
        grid=(_GN,),
        in_specs=[
            pl.BlockSpec((2, _BN, d_in), lambda i: (0, i, 0)),
            pl.BlockSpec((_BN, d_in), lambda i: (i, 0)),
            pl.BlockSpec((_BN, 1), lambda i: (i, 0)),
            pl.BlockSpec((1, d_in), lambda i: (0, 0)),
            pl.BlockSpec((d_in, d_out), lambda i: (0, 0)),
        ],
        out_specs=pl.BlockSpec((_BN, d_out), lambda i: (i, 0)),
        out_shape=jax.ShapeDtypeStruct((N_PAD, d_out), _f32),
    )(p, g, dinv, b, W)


def _tc_combine_only(p, g, dinv, b):
    d_in = g.shape[1]

    def body(p_ref, g_ref, dv_ref, b_ref, o_ref):
        dv = dv_ref[...]
        z = (p_ref[0] + p_ref[1] + g_ref[...]) * dv + b_ref[...]
        o_ref[...] = jnp.tanh(z) * dv

    return pl.pallas_call(
        body,
        grid=(_GN,),
        in_specs=[
            pl.BlockSpec((2, _BN, d_in), lambda i: (0, i, 0)),
            pl.BlockSpec((_BN, d_in), lambda i: (i, 0)),
            pl.BlockSpec((_BN, 1), lambda i: (i, 0)),
            pl.BlockSpec((1, d_in), lambda i: (0, 0)),
        ],
        out_specs=pl.BlockSpec((_BN, d_in), lambda i: (i, 0)),
        out_shape=jax.ShapeDtypeStruct((N_PAD, d_in), _f32),
    )(p, g, dinv, b)


def _tc_tail(p, g5, dinv, W5, b5, lW1p, lb1p, lW2p, lb2p, lW3p, lb3p):
    def body(p_ref, g_ref, dv_ref, w5_ref, b5_ref,
             w1_ref, c1_ref, w2_ref, c2_ref, w3_ref, c3_ref, o_ref):
        q = (p_ref[0] + p_ref[1] + g_ref[...]) * dv_ref[...]
        h5 = jnp.tanh(q @ w5_ref[...] + b5_ref[...])
        t1 = jnp.tanh(h5 @ w1_ref[...] + c1_ref[...])
        t2 = jnp.tanh(t1 @ w2_ref[...] + c2_ref[...])
        o_ref[...] = jnp.tanh(t2 @ w3_ref[...] + c3_ref[...])

    return pl.pallas_call(
        body,
        grid=(_GN,),
        in_specs=[
            pl.BlockSpec((2, _BN, 32), lambda i: (0, i, 0)),
            pl.BlockSpec((_BN, 32), lambda i: (i, 0)),
            pl.BlockSpec((_BN, 1), lambda i: (i, 0)),
            pl.BlockSpec((32, 256), lambda i: (0, 0)),
            pl.BlockSpec((1, 256), lambda i: (0, 0)),
            pl.BlockSpec((256, 32), lambda i: (0, 0)),
            pl.BlockSpec((1, 32), lambda i: (0, 0)),
            pl.BlockSpec((32, 32), lambda i: (0, 0)),
            pl.BlockSpec((1, 32), lambda i: (0, 0)),
            pl.BlockSpec((32, 16), lambda i: (0, 0)),
            pl.BlockSpec((1, 16), lambda i: (0, 0)),
        ],
        out_specs=pl.BlockSpec((_BN, 16), lambda i: (i, 0)),
        out_shape=jax.ShapeDtypeStruct((N_PAD, 16), _f32),
    )(p, g5, dinv, W5, b5, lW1p, lb1p, lW2p, lb2p, lW3p, lb3p)


_BE = 3200
_GE = E // _BE


def _tc_edge(he, Wout, Se, cbt):
    def body(hs_ref, hd_ref, wo_ref, se_ref, cb_ref, out_ref, e_ref):
        hsd = jnp.concatenate([hs_ref[0], hd_ref[0]], axis=1)
        dn = (((1,), (1,)), ((), ()))
        out_ref[...] = lax.dot_general(wo_ref[...], hsd, dn) + cb_ref[...]
        e_ref[...] = lax.dot_general(se_ref[...], hsd, dn)

    return pl.pallas_call(
        body,
        grid=(_GE,),
        in_specs=[
            pl.BlockSpec((1, _BE, 16), lambda i: (0, i, 0)),
            pl.BlockSpec((1, _BE, 16), lambda i: (1, i, 0)),
            pl.BlockSpec((40, 32), lambda i: (0, 0)),
            pl.BlockSpec((24, 32), lambda i: (0, 0)),
            pl.BlockSpec((40, 1), lambda i: (0, 0)),
        ],
        out_specs=[
            pl.BlockSpec((40, _BE), lambda i: (0, i)),
            pl.BlockSpec((24, _BE), lambda i: (0, i)),
        ],
        out_shape=[
            jax.ShapeDtypeStruct((40, E), _f32),
            jax.ShapeDtypeStruct((24, E), _f32),
        ],
    )(he, he, Wout, Se, cbt)


def kernel(x, edge_index, batch, W1, b1, W2, b2, W3, b3, W4, b4, W5, b5,
           lW1, lb1, lW2, lb2, lW3, lb3, cW, cb):
    pad_e = E_PAD - E
    src = jnp.concatenate(
        [edge_index[0], jnp.full((pad_e,), N_PAD - 1, _i32)]).reshape(NCH, CHUNK)
    dst = jnp.concatenate(
        [edge_index[1], jnp.full((pad_e,), N_PAD - 1, _i32)]).reshape(NCH, CHUNK)
    x_p = jnp.pad(x, ((0, N_PAD - N), (0, 0)))

    z16 = jnp.zeros((ZR, 16), _f32)
    z32 = jnp.zeros((ZR, 32), _f32)
    z64 = jnp.zeros((ZR, 64), _f32)
    e1 = jnp.zeros((N_PAD, 16), _f32).at[:, 0].set(1.0)

    lW1p = jnp.pad(lW1, ((0, 0), (0, 8)))
    lb1p = jnp.pad(lb1, (0, 8)).reshape(1, 32)
    lW2p = jnp.pad(lW2, ((0, 8), (0, 14)))
    lb2p = jnp.pad(lb2, (0, 14)).reshape(1, 32)
    lW3p = jnp.pad(lW3, ((0, 14), (0, 4)))
    lb3p = jnp.pad(lb3, (0, 4)).reshape(1, 16)

    cWa = jnp.pad(cW[:12], ((0, 4), (0, 0)))
    cWb = jnp.pad(cW[12:], ((0, 4), (0, 0)))
    Wout = jnp.concatenate([cWa.T, cWb.T], axis=1)
    ii = jnp.arange(12)
    Se = (jnp.zeros((24, 32), _f32)
          .at[ii, ii].set(1.0)
          .at[12 + ii, 16 + ii].set(1.0))
    cbt = cb.reshape(40, 1)

    degp = _sc_propagate(e1, z16, src, dst, d=16)

    dinv, g1 = _tc_head(degp, x_p, W1)
    p = _sc_propagate(g1, z64, src, dst, d=64)

    g2 = _tc_combine_transform(p, g1, dinv, b1.reshape(1, 64), W2)
    p = _sc_propagate(g2, z64, src, dst, d=64)
    g3 = _tc_combine_transform(p, g2, dinv, b2.reshape(1, 64), W3)
    p = _sc_propagate(g3, z32, src, dst, d=32)
    g4 = _tc_combine_transform(p, g3, dinv, b3.reshape(1, 32), W4)
    p = _sc_propagate(g4, z32, src, dst, d=32)

    g5 = _tc_combine_only(p, g4, dinv, b4.reshape(1, 32))
    p = _sc_propagate(g5, z32, src, dst, d=32)
    hf = _tc_tail(p, g5, dinv, W5, b5.reshape(1, 256),
                  lW1p, lb1p, lW2p, lb2p, lW3p, lb3p)

    he = _sc_edge_gather(hf, src, dst)
    out_t, e_t = _tc_edge(he, Wout, Se, cbt)
    return (out_t.T, e_t.T)
